# baseline (speedup 1.0000x reference)
import functools

import jax
import jax.numpy as jnp
from jax import lax
from jax.experimental import pallas as pl
from jax.experimental.pallas import tpu as pltpu
from jax.experimental.pallas import tpu_sc as plsc

B = 16384
D = 16
L = 16

_info = plsc.get_sparse_core_info()
NC, NS = _info.num_cores, _info.num_subcores
NW = NC * NS
BPW = B // NW
NBLK = BPW // L

_mesh = plsc.VectorSubcoreMesh(core_axis_name="c", subcore_axis_name="s")


@functools.partial(
    pl.kernel,
    out_type=jax.ShapeDtypeStruct((B,), jnp.float32),
    mesh=_mesh,
    compiler_params=pltpu.CompilerParams(
        needs_layout_passes=False, use_tc_tiling_on_sc=False),
    scratch_types=[
        pltpu.VMEM((BPW,), jnp.int32),
        pltpu.VMEM((BPW,), jnp.int32),
        pltpu.VMEM((BPW, D), jnp.float32),
        pltpu.VMEM((BPW, D), jnp.float32),
        pltpu.VMEM((BPW * D,), jnp.float32),
        pltpu.VMEM((BPW,), jnp.float32),
        pltpu.VMEM((L,), jnp.float32),
        pltpu.VMEM((L,), jnp.float32),
        pltpu.SemaphoreType.DMA,
        pltpu.SemaphoreType.DMA,
    ],
)
def _deepfm_sc(uid_hbm, tid_hbm, utab_hbm, itab_hbm, wv_hbm, bv_hbm, out_hbm,
               uidx_v, tidx_v, urows_v, trows_v, prod_v, out_v, wv, bv,
               usem, tsem):
    wid = lax.axis_index("s") * NC + lax.axis_index("c")
    base = wid * BPW

    pltpu.sync_copy(uid_hbm.at[pl.ds(base, BPW)], uidx_v)
    pltpu.sync_copy(tid_hbm.at[pl.ds(base, BPW)], tidx_v)
    ucp = pltpu.async_copy(utab_hbm.at[uidx_v], urows_v, usem)
    tcp = pltpu.async_copy(itab_hbm.at[tidx_v], trows_v, tsem)
    pltpu.sync_copy(wv_hbm, wv)
    pltpu.sync_copy(bv_hbm, bv)
    ucp.wait()
    tcp.wait()

    w = wv[...]
    bb = bv[...]
    lanes = lax.iota(jnp.int32, L)

    for r in range(BPW):
        prod_v[pl.ds(r * L, L)] = urows_v[r, :] * trows_v[r, :]

    stride_idx = lanes * D

    def block(i, _):
        acc = jnp.zeros((L,), jnp.float32)
        for d in range(D):
            acc = acc + plsc.load_gather(prod_v, [i * (L * D) + d + stride_idx])
        x = acc * w + bb
        y = 1.0 / (1.0 + jnp.exp(-x))
        out_v[pl.ds(i * L, L)] = y
        return 0

    lax.fori_loop(0, NBLK, block, 0)
    pltpu.sync_copy(out_v, out_hbm.at[pl.ds(base, BPW)])


def kernel(uid_idx, tid_idx, user_table, item_table, W, b):
    uid = uid_idx.astype(jnp.int32)
    tid = tid_idx.astype(jnp.int32)
    wv = jnp.broadcast_to(W.reshape(()), (L,)).astype(jnp.float32)
    bv = jnp.broadcast_to(b.reshape(()), (L,)).astype(jnp.float32)
    y = _deepfm_sc(uid, tid, user_table, item_table, wv, bv)
    return y.reshape(B, 1)


# speedup vs baseline: 1.0005x; 1.0005x over previous
import functools

import jax
import jax.numpy as jnp
from jax import lax
from jax.experimental import pallas as pl
from jax.experimental.pallas import tpu as pltpu
from jax.experimental.pallas import tpu_sc as plsc

B = 16384
D = 16
L = 16

_info = plsc.get_sparse_core_info()
NC, NS = _info.num_cores, _info.num_subcores
NW = NC * NS
BPW = B // NW
NBLK = BPW // L

_mesh = plsc.VectorSubcoreMesh(core_axis_name="c", subcore_axis_name="s")


@functools.partial(
    pl.kernel,
    out_type=jax.ShapeDtypeStruct((B,), jnp.float32),
    mesh=_mesh,
    compiler_params=pltpu.CompilerParams(
        needs_layout_passes=False, use_tc_tiling_on_sc=False),
    scratch_types=[
        pltpu.VMEM((BPW,), jnp.int32),
        pltpu.VMEM((BPW,), jnp.int32),
        pltpu.VMEM((BPW, D), jnp.float32),
        pltpu.VMEM((BPW, D), jnp.float32),
        pltpu.VMEM((BPW,), jnp.float32),
        pltpu.VMEM((L,), jnp.float32),
        pltpu.VMEM((L,), jnp.float32),
        pltpu.SemaphoreType.DMA,
        pltpu.SemaphoreType.DMA,
    ],
)
def _deepfm_sc(uid_hbm, tid_hbm, utab_hbm, itab_hbm, wv_hbm, bv_hbm, out_hbm,
               uidx_v, tidx_v, urows_v, trows_v, out_v, wv, bv,
               usem, tsem):
    wid = lax.axis_index("s") * NC + lax.axis_index("c")
    base = wid * BPW

    pltpu.sync_copy(uid_hbm.at[pl.ds(base, BPW)], uidx_v)
    pltpu.sync_copy(tid_hbm.at[pl.ds(base, BPW)], tidx_v)
    ucp = pltpu.async_copy(utab_hbm.at[uidx_v], urows_v, usem)
    tcp = pltpu.async_copy(itab_hbm.at[tidx_v], trows_v, tsem)
    pltpu.sync_copy(wv_hbm, wv)
    pltpu.sync_copy(bv_hbm, bv)
    ucp.wait()
    tcp.wait()

    w = wv[...]
    bb = bv[...]
    lanes = lax.iota(jnp.int32, L)

    def block(i, _):
        rows = i * L + lanes
        acc = jnp.zeros((L,), jnp.float32)
        for d in range(D):
            col = jnp.full((L,), d, jnp.int32)
            u = plsc.load_gather(urows_v, [rows, col])
            t = plsc.load_gather(trows_v, [rows, col])
            acc = acc + u * t
        x = acc * w + bb
        y = 1.0 / (1.0 + jnp.exp(-x))
        out_v[pl.ds(i * L, L)] = y
        return 0

    lax.fori_loop(0, NBLK, block, 0)
    pltpu.sync_copy(out_v, out_hbm.at[pl.ds(base, BPW)])


def kernel(uid_idx, tid_idx, user_table, item_table, W, b):
    uid = uid_idx.astype(jnp.int32)
    tid = tid_idx.astype(jnp.int32)
    wv = jnp.broadcast_to(W.reshape(()), (L,)).astype(jnp.float32)
    bv = jnp.broadcast_to(b.reshape(()), (L,)).astype(jnp.float32)
    y = _deepfm_sc(uid, tid, user_table, item_table, wv, bv)
    return y.reshape(B, 1)


# speedup vs baseline: 5.9506x; 5.9477x over previous
import functools

import jax
import jax.numpy as jnp
from jax import lax
from jax.experimental import pallas as pl
from jax.experimental.pallas import tpu as pltpu
from jax.experimental.pallas import tpu_sc as plsc

B = 16384
V = 1000000
D = 16
L = 16

_info = plsc.get_sparse_core_info()
NC, NS = _info.num_cores, _info.num_subcores
NW = NC * NS
BPW = B // NW
NBLK = BPW // L

CK = 32768
NFULL = V // CK
TAIL = V - NFULL * CK
VCUT = (V // 128) * 128
NT = V - VCUT

_mesh = plsc.VectorSubcoreMesh(core_axis_name="c", subcore_axis_name="s")


@functools.partial(
    pl.kernel,
    out_type=[
        jax.ShapeDtypeStruct((V * D,), jnp.float32),
        jax.ShapeDtypeStruct((V * D,), jnp.float32),
    ],
    mesh=_mesh,
    compiler_params=pltpu.CompilerParams(use_tc_tiling_on_sc=True),
    scratch_types=[
        pltpu.VMEM((CK,), jnp.float32),
        pltpu.VMEM((CK,), jnp.float32),
        pltpu.SemaphoreType.DMA,
        pltpu.SemaphoreType.DMA,
        pltpu.SemaphoreType.DMA,
        pltpu.SemaphoreType.DMA,
    ],
)
def _detile_sc(utab_hbm, itab_hbm, ou_hbm, oi_hbm,
               bufa, bufb, rsa, rsb, wsa, wsb):
    wid = lax.axis_index("s") * NC + lax.axis_index("c")
    d = wid % D
    base = d * V

    def stream(tab, out):
        bufs = (bufa, bufb)
        rsems = (rsa, rsb)
        wsems = (wsa, wsb)
        rd = {}
        wr = {}
        rd[0] = pltpu.async_copy(tab.at[d, pl.ds(0, CK)], bufs[0], rsems[0])
        for i in range(NFULL):
            b = i & 1
            rd[i].wait()
            if i >= 1:
                wr[i - 1].wait()
            if i + 1 < NFULL:
                nb = (i + 1) & 1
                rd[i + 1] = pltpu.async_copy(
                    tab.at[d, pl.ds((i + 1) * CK, CK)], bufs[nb], rsems[nb])
            wr[i] = pltpu.async_copy(
                bufs[b], out.at[pl.ds(base + i * CK, CK)], wsems[b])
        wr[NFULL - 1].wait()
        t1 = (TAIL // 128) * 128
        pltpu.sync_copy(tab.at[d, pl.ds(NFULL * CK, t1)],
                        bufs[0].at[pl.ds(0, t1)])
        pltpu.sync_copy(bufs[0].at[pl.ds(0, t1)],
                        out.at[pl.ds(base + NFULL * CK, t1)])

    @pl.when(wid < D)
    def _():
        stream(utab_hbm, ou_hbm)

    @pl.when(wid >= D)
    def _():
        stream(itab_hbm, oi_hbm)


@functools.partial(
    pl.kernel,
    out_type=jax.ShapeDtypeStruct((B,), jnp.float32),
    mesh=_mesh,
    compiler_params=pltpu.CompilerParams(
        use_tc_tiling_on_sc=False, needs_layout_passes=False),
    scratch_types=[
        pltpu.VMEM((BPW,), jnp.int32),
        pltpu.VMEM((BPW,), jnp.int32),
        pltpu.VMEM((D, BPW), jnp.float32),
        pltpu.VMEM((D, BPW), jnp.float32),
        pltpu.VMEM((NT * D,), jnp.float32),
        pltpu.VMEM((NT * D,), jnp.float32),
        pltpu.VMEM((BPW,), jnp.float32),
        pltpu.VMEM((L,), jnp.float32),
        pltpu.VMEM((L,), jnp.float32),
        pltpu.SemaphoreType.DMA,
        pltpu.SemaphoreType.DMA,
    ],
)
def _gather_sc(uid_hbm, tid_hbm, ulin_hbm, ilin_hbm, utail_hbm, itail_hbm,
               wv_hbm, bv_hbm, out_hbm,
               uidx_v, tidx_v, urows_v, trows_v, utail_v, itail_v,
               out_v, wv, bv, usem, tsem):
    wid = lax.axis_index("s") * NC + lax.axis_index("c")
    base = wid * BPW

    pltpu.sync_copy(uid_hbm.at[pl.ds(base, BPW)], uidx_v)
    pltpu.sync_copy(tid_hbm.at[pl.ds(base, BPW)], tidx_v)
    ucps = [
        pltpu.async_copy(ulin_hbm.at[pl.ds(d * V, V)].at[uidx_v],
                         urows_v.at[d], usem)
        for d in range(D)
    ]
    tcps = [
        pltpu.async_copy(ilin_hbm.at[pl.ds(d * V, V)].at[tidx_v],
                         trows_v.at[d], tsem)
        for d in range(D)
    ]
    pltpu.sync_copy(wv_hbm, wv)
    pltpu.sync_copy(bv_hbm, bv)
    pltpu.sync_copy(utail_hbm, utail_v)
    pltpu.sync_copy(itail_hbm, itail_v)
    for cp in ucps:
        cp.wait()
    for cp in tcps:
        cp.wait()

    w = wv[...]
    bb = bv[...]

    def block(i, _):
        sl = pl.ds(i * L, L)
        uidx = uidx_v[sl]
        tidx = tidx_v[sl]
        um = uidx >= VCUT
        tm = tidx >= VCUT
        ut = jnp.clip(uidx - VCUT, 0, NT - 1) * D
        tt = jnp.clip(tidx - VCUT, 0, NT - 1) * D
        acc = jnp.zeros((L,), jnp.float32)
        for d in range(D):
            u = jnp.where(um, plsc.load_gather(utail_v, [ut + d]),
                          urows_v[d, sl])
            t = jnp.where(tm, plsc.load_gather(itail_v, [tt + d]),
                          trows_v[d, sl])
            acc = acc + u * t
        x = acc * w + bb
        y = 1.0 / (1.0 + jnp.exp(-x))
        out_v[sl] = y
        return 0

    lax.fori_loop(0, NBLK, block, 0)
    pltpu.sync_copy(out_v, out_hbm.at[pl.ds(base, BPW)])


def kernel(uid_idx, tid_idx, user_table, item_table, W, b):
    uid = uid_idx.astype(jnp.int32)
    tid = tid_idx.astype(jnp.int32)
    wv = jnp.broadcast_to(W.reshape(()), (L,)).astype(jnp.float32)
    bv = jnp.broadcast_to(b.reshape(()), (L,)).astype(jnp.float32)
    utail = user_table[VCUT:, :].reshape(NT * D)
    itail = item_table[VCUT:, :].reshape(NT * D)
    ulin, ilin = _detile_sc(user_table.T, item_table.T)
    y = _gather_sc(uid, tid, ulin, ilin, utail, itail, wv, bv)
    return y.reshape(B, 1)


# speedup vs baseline: 6.0656x; 1.0193x over previous
import functools

import jax
import jax.numpy as jnp
from jax import lax
from jax.experimental import pallas as pl
from jax.experimental.pallas import tpu as pltpu
from jax.experimental.pallas import tpu_sc as plsc

B = 16384
V = 1000000
D = 16
L = 16

_info = plsc.get_sparse_core_info()
NC, NS = _info.num_cores, _info.num_subcores
NW = NC * NS
BPW = B // NW
NBLK = BPW // L

CK = 61440
NFULL = V // CK
TAIL = V - NFULL * CK
VCUT = (V // 128) * 128
NT = V - VCUT

_mesh = plsc.VectorSubcoreMesh(core_axis_name="c", subcore_axis_name="s")


@functools.partial(
    pl.kernel,
    out_type=[
        jax.ShapeDtypeStruct((V * D,), jnp.float32),
        jax.ShapeDtypeStruct((V * D,), jnp.float32),
    ],
    mesh=_mesh,
    compiler_params=pltpu.CompilerParams(use_tc_tiling_on_sc=True),
    scratch_types=[
        pltpu.VMEM((CK,), jnp.float32),
        pltpu.VMEM((CK,), jnp.float32),
        pltpu.SemaphoreType.DMA,
        pltpu.SemaphoreType.DMA,
        pltpu.SemaphoreType.DMA,
        pltpu.SemaphoreType.DMA,
    ],
)
def _detile_sc(utab_hbm, itab_hbm, ou_hbm, oi_hbm,
               bufa, bufb, rsa, rsb, wsa, wsb):
    wid = lax.axis_index("s") * NC + lax.axis_index("c")
    d = wid % D
    base = d * V

    def stream(tab, out):
        bufs = (bufa, bufb)
        rsems = (rsa, rsb)
        wsems = (wsa, wsb)
        rd = {}
        wr = {}
        rd[0] = pltpu.async_copy(tab.at[d, pl.ds(0, CK)], bufs[0], rsems[0])
        for i in range(NFULL):
            b = i & 1
            rd[i].wait()
            if i >= 1:
                wr[i - 1].wait()
            if i + 1 < NFULL:
                nb = (i + 1) & 1
                rd[i + 1] = pltpu.async_copy(
                    tab.at[d, pl.ds((i + 1) * CK, CK)], bufs[nb], rsems[nb])
            wr[i] = pltpu.async_copy(
                bufs[b], out.at[pl.ds(base + i * CK, CK)], wsems[b])
        wr[NFULL - 1].wait()
        t1 = (TAIL // 128) * 128
        pltpu.sync_copy(tab.at[d, pl.ds(NFULL * CK, t1)],
                        bufs[0].at[pl.ds(0, t1)])
        pltpu.sync_copy(bufs[0].at[pl.ds(0, t1)],
                        out.at[pl.ds(base + NFULL * CK, t1)])

    @pl.when(wid < D)
    def _():
        stream(utab_hbm, ou_hbm)

    @pl.when(wid >= D)
    def _():
        stream(itab_hbm, oi_hbm)


@functools.partial(
    pl.kernel,
    out_type=jax.ShapeDtypeStruct((B,), jnp.float32),
    mesh=_mesh,
    compiler_params=pltpu.CompilerParams(
        use_tc_tiling_on_sc=False, needs_layout_passes=False),
    scratch_types=[
        pltpu.VMEM((BPW,), jnp.int32),
        pltpu.VMEM((BPW,), jnp.int32),
        pltpu.VMEM((D, BPW), jnp.float32),
        pltpu.VMEM((D, BPW), jnp.float32),
        pltpu.VMEM((NT * D,), jnp.float32),
        pltpu.VMEM((NT * D,), jnp.float32),
        pltpu.VMEM((BPW,), jnp.float32),
        pltpu.VMEM((L,), jnp.float32),
        pltpu.VMEM((L,), jnp.float32),
        pltpu.SemaphoreType.DMA,
        pltpu.SemaphoreType.DMA,
    ],
)
def _gather_sc(uid_hbm, tid_hbm, ulin_hbm, ilin_hbm, utail_hbm, itail_hbm,
               wv_hbm, bv_hbm, out_hbm,
               uidx_v, tidx_v, urows_v, trows_v, utail_v, itail_v,
               out_v, wv, bv, usem, tsem):
    wid = lax.axis_index("s") * NC + lax.axis_index("c")
    base = wid * BPW

    pltpu.sync_copy(uid_hbm.at[pl.ds(base, BPW)], uidx_v)
    pltpu.sync_copy(tid_hbm.at[pl.ds(base, BPW)], tidx_v)
    ucps = [
        pltpu.async_copy(ulin_hbm.at[pl.ds(d * V, V)].at[uidx_v],
                         urows_v.at[d], usem)
        for d in range(D)
    ]
    tcps = [
        pltpu.async_copy(ilin_hbm.at[pl.ds(d * V, V)].at[tidx_v],
                         trows_v.at[d], tsem)
        for d in range(D)
    ]
    pltpu.sync_copy(wv_hbm, wv)
    pltpu.sync_copy(bv_hbm, bv)
    pltpu.sync_copy(utail_hbm, utail_v)
    pltpu.sync_copy(itail_hbm, itail_v)
    for cp in ucps:
        cp.wait()
    for cp in tcps:
        cp.wait()

    w = wv[...]
    bb = bv[...]

    def block(i, _):
        sl = pl.ds(i * L, L)
        uidx = uidx_v[sl]
        tidx = tidx_v[sl]
        um = uidx >= VCUT
        tm = tidx >= VCUT
        ut = jnp.clip(uidx - VCUT, 0, NT - 1) * D
        tt = jnp.clip(tidx - VCUT, 0, NT - 1) * D
        acc = jnp.zeros((L,), jnp.float32)
        for d in range(D):
            u = jnp.where(um, plsc.load_gather(utail_v, [ut + d]),
                          urows_v[d, sl])
            t = jnp.where(tm, plsc.load_gather(itail_v, [tt + d]),
                          trows_v[d, sl])
            acc = acc + u * t
        x = acc * w + bb
        y = 1.0 / (1.0 + jnp.exp(-x))
        out_v[sl] = y
        return 0

    lax.fori_loop(0, NBLK, block, 0)
    pltpu.sync_copy(out_v, out_hbm.at[pl.ds(base, BPW)])


def kernel(uid_idx, tid_idx, user_table, item_table, W, b):
    uid = uid_idx.astype(jnp.int32)
    tid = tid_idx.astype(jnp.int32)
    wv = jnp.broadcast_to(W.reshape(()), (L,)).astype(jnp.float32)
    bv = jnp.broadcast_to(b.reshape(()), (L,)).astype(jnp.float32)
    utail = user_table[VCUT:, :].reshape(NT * D)
    itail = item_table[VCUT:, :].reshape(NT * D)
    ulin, ilin = _detile_sc(user_table.T, item_table.T)
    y = _gather_sc(uid, tid, ulin, ilin, utail, itail, wv, bv)
    return y.reshape(B, 1)
